# baseline (speedup 1.0000x reference)
import functools

import jax
import jax.numpy as jnp
from jax.experimental import pallas as pl
from jax.experimental.pallas import tpu as pltpu

IN_C, IN_H, IN_W = 4, 84, 84
C1_OC, C2_OC, C3_OC = 16, 32, 32
HID = 256
HEAD_N = 128
NA = 6
BB = 8
FEAT = C3_OC * 49
HQ = IN_H // 4
HQP = 22
WQP = 24


SUB = 4


def _conv_kernel(x_ref, w1_ref, b1_ref, w2_ref, b2_ref, w3_ref, b3_ref,
                 out_ref):
    for s in range(SUB):
        _conv_block(x_ref[s], w1_ref, b1_ref, w2_ref, b2_ref, w3_ref, b3_ref,
                    out_ref, s)


def _conv_block(xs, w1_ref, b1_ref, w2_ref, b2_ref, w3_ref, b3_ref,
                out_ref, s):
    f32 = jnp.float32
    bf16 = jnp.bfloat16

    o1 = jnp.dot(xs, w1_ref[...], preferred_element_type=f32)
    o1 = o1.reshape(HQP // 2, 2, WQP // 2, 2, 4 * BB * C1_OC)
    phases = []
    for ry in range(2):
        for rx in range(2):
            acc = None
            for a in range(2):
                for b2 in range(2):
                    d, e = ry + a, rx + b2
                    lo = (a * 2 + b2) * 128
                    w = o1[d // 2:d // 2 + 10, d % 2,
                           e // 2:e // 2 + 10, e % 2, lo:lo + 128]
                    acc = w if acc is None else acc + w
            phases.append(jnp.maximum(acc + b1_ref[...], 0.0).astype(bf16))
    h1m = jnp.concatenate(phases, axis=-1)

    o2 = jnp.dot(h1m.reshape(100, 4 * BB * C1_OC), w2_ref[...],
                 preferred_element_type=f32)
    o2 = o2.reshape(10, 10, 4 * BB * C2_OC)
    h2 = (o2[0:9, 0:9, 0:256] + o2[0:9, 1:10, 256:512] +
          o2[1:10, 0:9, 512:768] + o2[1:10, 1:10, 768:1024])
    h2 = jnp.maximum(h2 + b2_ref[...], 0.0).astype(bf16)

    o3 = jnp.dot(h2.reshape(81, BB * C2_OC), w3_ref[...],
                 preferred_element_type=f32)
    o3 = o3.reshape(9, 9, 9 * BB * C3_OC)
    h3 = o3[0:7, 0:7, 0:256]
    for t in range(1, 9):
        i, j = t // 3, t % 3
        h3 = h3 + o3[i:i + 7, j:j + 7, t * 256:(t + 1) * 256]
    h3 = jnp.maximum(h3 + b3_ref[...], 0.0).astype(bf16)

    t3 = h3.reshape(49, BB * C3_OC).T
    for c in range(C3_OC):
        out_ref[s * BB:(s + 1) * BB, c * 49:(c + 1) * 49] = \
            t3[c * BB:(c + 1) * BB, :]


def _head_kernel(f_ref, wh_ref, bh_ref, wo_ref, bo_ref, out_ref):
    f32 = jnp.float32
    hh = jnp.dot(f_ref[...], wh_ref[...], preferred_element_type=f32)
    hh = jnp.maximum(hh + bh_ref[...], 0.0).astype(jnp.bfloat16)
    out = jnp.dot(hh, wo_ref[...], preferred_element_type=f32) + bo_ref[...]
    lane = jax.lax.broadcasted_iota(jnp.int32, out.shape, 1)
    logits = jnp.where(lane < NA, out, -1e30)
    m = jnp.max(logits, axis=-1, keepdims=True)
    e = jnp.exp(logits - m)
    probs = e / jnp.sum(e, axis=-1, keepdims=True)
    out_ref[...] = jnp.where(lane == NA, out, probs)


def kernel(w1, b1, sel2, w2, b2, sel3, w3, b3, whead, bhead, wout, bout, x):
    del sel2, sel3
    f32, bf16 = jnp.float32, jnp.bfloat16
    B = x.shape[0]
    nblk = -(-B // (BB * SUB)) * SUB
    Bp = nblk * BB

    xb = x
    if Bp != B:
        xb = jnp.pad(xb, ((0, Bp - B), (0, 0), (0, 0), (0, 0)))

    xa = xb.reshape(nblk, BB, IN_C, HQ, 4, IN_W)
    xa = jnp.transpose(xa, (0, 3, 1, 2, 4, 5)).astype(bf16)
    xa = jnp.pad(xa, ((0, 0), (0, HQP - HQ), (0, 0), (0, 0), (0, 0),
                      (0, 4 * WQP - IN_W)))
    xa = jax.lax.optimization_barrier(xa)
    xa = xa.reshape(nblk, HQP, BB, IN_C, 4, WQP, 4)
    xph = jnp.transpose(xa, (0, 1, 5, 6, 2, 3, 4))
    xph = xph.reshape(nblk, HQP * WQP, BB * IN_C * 16)

    w1c = w1.reshape(BB, IN_C, 2, 4, 2, 4, BB * C1_OC)
    w1c = w1c.transpose(5, 0, 1, 3, 2, 4, 6).reshape(BB * IN_C * 16,
                                                     4 * BB * C1_OC)

    tidx = jnp.array([[[[(2 * a + ry) * 4 + (2 * b + rx)
                         for b in range(2)] for a in range(2)]
                       for rx in range(2)] for ry in range(2)], jnp.int32)
    w2c = w2[tidx.reshape(-1)].reshape(2, 2, 2, 2, BB * C1_OC, BB * C2_OC)
    w2c = w2c.transpose(0, 1, 4, 2, 3, 5).reshape(4 * BB * C1_OC,
                                                  4 * BB * C2_OC)

    w3c = w3.transpose(1, 0, 2).reshape(BB * C2_OC, 9 * BB * C3_OC)

    feat = pl.pallas_call(
        _conv_kernel,
        out_shape=jax.ShapeDtypeStruct((Bp, FEAT), bf16),
        grid=(nblk // SUB,),
        in_specs=[
            pl.BlockSpec((SUB, HQP * WQP, BB * IN_C * 16), lambda i: (i, 0, 0)),
            pl.BlockSpec(w1c.shape, lambda i: (0, 0)),
            pl.BlockSpec(b1.shape, lambda i: (0, 0)),
            pl.BlockSpec(w2c.shape, lambda i: (0, 0)),
            pl.BlockSpec(b2.shape, lambda i: (0, 0)),
            pl.BlockSpec(w3c.shape, lambda i: (0, 0)),
            pl.BlockSpec(b3.shape, lambda i: (0, 0)),
        ],
        out_specs=pl.BlockSpec((SUB * BB, FEAT), lambda i: (i, 0)),
        compiler_params=pltpu.CompilerParams(
            dimension_semantics=("parallel",),
            vmem_limit_bytes=64 * 1024 * 1024),
    )(xph, w1c, b1, w2c, b2, w3c, b3)

    HB = 128
    nhb = -(-Bp // HB)
    Bh = nhb * HB
    if Bh != Bp:
        feat = jnp.pad(feat, ((0, Bh - Bp), (0, 0)))
    y = pl.pallas_call(
        _head_kernel,
        out_shape=jax.ShapeDtypeStruct((Bh, HEAD_N), f32),
        grid=(nhb,),
        in_specs=[
            pl.BlockSpec((HB, FEAT), lambda i: (i, 0)),
            pl.BlockSpec(whead.shape, lambda i: (0, 0)),
            pl.BlockSpec(bhead.shape, lambda i: (0, 0)),
            pl.BlockSpec(wout.shape, lambda i: (0, 0)),
            pl.BlockSpec(bout.shape, lambda i: (0, 0)),
        ],
        out_specs=pl.BlockSpec((HB, HEAD_N), lambda i: (i, 0)),
        compiler_params=pltpu.CompilerParams(
            dimension_semantics=("parallel",),
            vmem_limit_bytes=64 * 1024 * 1024),
    )(feat, whead, bhead, wout, bout)

    probs = y[:B, :NA]
    value = y[:B, NA:NA + 1]
    return probs, value


# speedup vs baseline: 1.0471x; 1.0471x over previous
import functools

import jax
import jax.numpy as jnp
from jax.experimental import pallas as pl
from jax.experimental.pallas import tpu as pltpu

IN_C, IN_H, IN_W = 4, 84, 84
C1_OC, C2_OC, C3_OC = 16, 32, 32
HID = 256
HEAD_N = 128
NA = 6
BB = 8
FEAT = C3_OC * 49
HQ = IN_H // 4
HQP = 22
WQP = 24


SUB = 4


def _conv_kernel(x_ref, w1_ref, b1_ref, w2_ref, b2_ref, w3_ref, b3_ref,
                 out_ref):
    for s in range(SUB):
        _conv_block(x_ref[s], w1_ref, b1_ref, w2_ref, b2_ref, w3_ref, b3_ref,
                    out_ref, s)


def _conv_block(xs, w1_ref, b1_ref, w2_ref, b2_ref, w3_ref, b3_ref,
                out_ref, s):
    f32 = jnp.float32
    bf16 = jnp.bfloat16

    o1 = jnp.dot(xs, w1_ref[...], preferred_element_type=f32)
    o1 = o1.reshape(HQP, WQP, 4 * BB * C1_OC)
    h1 = (o1[0:20, 0:20, 0:128] + o1[0:20, 1:21, 128:256] +
          o1[1:21, 0:20, 256:384] + o1[1:21, 1:21, 384:512])
    h1 = jnp.maximum(h1 + b1_ref[...], 0.0).astype(bf16)

    h1p = h1.reshape(10, 2, 10, 2, BB * C1_OC)
    o2 = None
    for p, (ry, rx) in enumerate(((0, 0), (0, 1), (1, 0), (1, 1))):
        g = h1p[:, ry, :, rx, :].reshape(100, BB * C1_OC)
        d = jnp.dot(g, w2_ref[p], preferred_element_type=f32)
        o2 = d if o2 is None else o2 + d
    o2 = o2.reshape(10, 10, 4 * BB * C2_OC)
    h2 = (o2[0:9, 0:9, 0:256] + o2[0:9, 1:10, 256:512] +
          o2[1:10, 0:9, 512:768] + o2[1:10, 1:10, 768:1024])
    h2 = jnp.maximum(h2 + b2_ref[...], 0.0).astype(bf16)

    o3 = jnp.dot(h2.reshape(81, BB * C2_OC), w3_ref[...],
                 preferred_element_type=f32)
    o3 = o3.reshape(9, 9, 9 * BB * C3_OC)
    h3 = o3[0:7, 0:7, 0:256]
    for t in range(1, 9):
        i, j = t // 3, t % 3
        h3 = h3 + o3[i:i + 7, j:j + 7, t * 256:(t + 1) * 256]
    h3 = jnp.maximum(h3 + b3_ref[...], 0.0).astype(bf16)

    t3 = h3.reshape(49, BB * C3_OC).T
    for c in range(C3_OC):
        out_ref[s * BB:(s + 1) * BB, c * 49:(c + 1) * 49] = \
            t3[c * BB:(c + 1) * BB, :]


def _head_kernel(f_ref, wh_ref, bh_ref, wo_ref, bo_ref, out_ref):
    f32 = jnp.float32
    hh = jnp.dot(f_ref[...], wh_ref[...], preferred_element_type=f32)
    hh = jnp.maximum(hh + bh_ref[...], 0.0).astype(jnp.bfloat16)
    out = jnp.dot(hh, wo_ref[...], preferred_element_type=f32) + bo_ref[...]
    lane = jax.lax.broadcasted_iota(jnp.int32, out.shape, 1)
    logits = jnp.where(lane < NA, out, -1e30)
    m = jnp.max(logits, axis=-1, keepdims=True)
    e = jnp.exp(logits - m)
    probs = e / jnp.sum(e, axis=-1, keepdims=True)
    out_ref[...] = jnp.where(lane == NA, out, probs)


def kernel(w1, b1, sel2, w2, b2, sel3, w3, b3, whead, bhead, wout, bout, x):
    del sel2, sel3
    f32, bf16 = jnp.float32, jnp.bfloat16
    B = x.shape[0]
    nblk = -(-B // (BB * SUB)) * SUB
    Bp = nblk * BB

    xb = x
    if Bp != B:
        xb = jnp.pad(xb, ((0, Bp - B), (0, 0), (0, 0), (0, 0)))

    xa = xb.reshape(nblk, BB, IN_C, HQ, 4, IN_W)
    xa = jnp.transpose(xa, (0, 3, 1, 2, 4, 5)).astype(bf16)
    xa = jnp.pad(xa, ((0, 0), (0, HQP - HQ), (0, 0), (0, 0), (0, 0),
                      (0, 4 * WQP - IN_W)))
    xa = jax.lax.optimization_barrier(xa)
    xa = xa.reshape(nblk, HQP, BB, IN_C, 4, WQP, 4)
    xph = jnp.transpose(xa, (0, 1, 5, 6, 2, 3, 4))
    xph = xph.reshape(nblk, HQP * WQP, BB * IN_C * 16)

    w1c = w1.reshape(BB, IN_C, 2, 4, 2, 4, BB * C1_OC)
    w1c = w1c.transpose(5, 0, 1, 3, 2, 4, 6).reshape(BB * IN_C * 16,
                                                     4 * BB * C1_OC)

    tidx = jnp.array([[[[(2 * a + ry) * 4 + (2 * b + rx)
                         for b in range(2)] for a in range(2)]
                       for rx in range(2)] for ry in range(2)], jnp.int32)
    w2c = w2[tidx.reshape(-1)].reshape(2, 2, 2, 2, BB * C1_OC, BB * C2_OC)
    w2c = w2c.transpose(0, 1, 4, 2, 3, 5).reshape(4, BB * C1_OC,
                                                  4 * BB * C2_OC)

    w3c = w3.transpose(1, 0, 2).reshape(BB * C2_OC, 9 * BB * C3_OC)

    feat = pl.pallas_call(
        _conv_kernel,
        out_shape=jax.ShapeDtypeStruct((Bp, FEAT), bf16),
        grid=(nblk // SUB,),
        in_specs=[
            pl.BlockSpec((SUB, HQP * WQP, BB * IN_C * 16), lambda i: (i, 0, 0)),
            pl.BlockSpec(w1c.shape, lambda i: (0, 0)),
            pl.BlockSpec(b1.shape, lambda i: (0, 0)),
            pl.BlockSpec(w2c.shape, lambda i: (0, 0, 0)),
            pl.BlockSpec(b2.shape, lambda i: (0, 0)),
            pl.BlockSpec(w3c.shape, lambda i: (0, 0)),
            pl.BlockSpec(b3.shape, lambda i: (0, 0)),
        ],
        out_specs=pl.BlockSpec((SUB * BB, FEAT), lambda i: (i, 0)),
        compiler_params=pltpu.CompilerParams(
            dimension_semantics=("parallel",),
            vmem_limit_bytes=64 * 1024 * 1024),
    )(xph, w1c, b1, w2c, b2, w3c, b3)

    HB = 128
    nhb = -(-Bp // HB)
    Bh = nhb * HB
    if Bh != Bp:
        feat = jnp.pad(feat, ((0, Bh - Bp), (0, 0)))
    y = pl.pallas_call(
        _head_kernel,
        out_shape=jax.ShapeDtypeStruct((Bh, HEAD_N), f32),
        grid=(nhb,),
        in_specs=[
            pl.BlockSpec((HB, FEAT), lambda i: (i, 0)),
            pl.BlockSpec(whead.shape, lambda i: (0, 0)),
            pl.BlockSpec(bhead.shape, lambda i: (0, 0)),
            pl.BlockSpec(wout.shape, lambda i: (0, 0)),
            pl.BlockSpec(bout.shape, lambda i: (0, 0)),
        ],
        out_specs=pl.BlockSpec((HB, HEAD_N), lambda i: (i, 0)),
        compiler_params=pltpu.CompilerParams(
            dimension_semantics=("parallel",),
            vmem_limit_bytes=64 * 1024 * 1024),
    )(feat, whead, bhead, wout, bout)

    probs = y[:B, :NA]
    value = y[:B, NA:NA + 1]
    return probs, value


# speedup vs baseline: 1.2818x; 1.2242x over previous
import jax
import jax.numpy as jnp
from jax.experimental import pallas as pl
from jax.experimental.pallas import tpu as pltpu

IN_C, IN_H, IN_W = 4, 84, 84
C1_OC, C2_OC, C3_OC = 16, 32, 32
HID = 256
HEAD_N = 128
NA = 6
BB = 8
FEAT = C3_OC * 49
HQ = IN_H // 4
HQP = 22
WQP = 24


SUB = 4


def _conv_kernel(x_ref, w1_ref, b1_ref, w2_ref, b2_ref, w3_ref, b3_ref,
                 out_ref):
    for s in range(SUB):
        _conv_block(x_ref[s], w1_ref, b1_ref, w2_ref, b2_ref, w3_ref, b3_ref,
                    out_ref, s)


def _conv_block(xs, w1_ref, b1_ref, w2_ref, b2_ref, w3_ref, b3_ref,
                out_ref, s):
    f32 = jnp.float32
    bf16 = jnp.bfloat16

    xt = jnp.swapaxes(xs, 1, 2)
    xt = xt.reshape(HQP, WQP, 4, BB * IN_C * 4)
    o1 = None
    for wr in range(4):
        g = xt[:, :, wr, :].reshape(HQP * WQP, BB * IN_C * 4)
        d = jnp.dot(g, w1_ref[wr], preferred_element_type=f32)
        o1 = d if o1 is None else o1 + d
    o1 = o1.reshape(HQP, WQP, 4 * BB * C1_OC)
    h1 = (o1[0:20, 0:20, 0:128] + o1[0:20, 1:21, 128:256] +
          o1[1:21, 0:20, 256:384] + o1[1:21, 1:21, 384:512])
    h1 = jnp.maximum(h1 + b1_ref[...], 0.0).astype(bf16)

    h1p = h1.reshape(10, 2, 10, 2, BB * C1_OC)
    o2 = None
    for p, (ry, rx) in enumerate(((0, 0), (0, 1), (1, 0), (1, 1))):
        g = h1p[:, ry, :, rx, :].reshape(100, BB * C1_OC)
        d = jnp.dot(g, w2_ref[p], preferred_element_type=f32)
        o2 = d if o2 is None else o2 + d
    o2 = o2.reshape(10, 10, 4 * BB * C2_OC)
    h2 = (o2[0:9, 0:9, 0:256] + o2[0:9, 1:10, 256:512] +
          o2[1:10, 0:9, 512:768] + o2[1:10, 1:10, 768:1024])
    h2 = jnp.maximum(h2 + b2_ref[...], 0.0).astype(bf16)

    o3 = jnp.dot(h2.reshape(81, BB * C2_OC), w3_ref[...],
                 preferred_element_type=f32)
    o3 = o3.reshape(9, 9, 9 * BB * C3_OC)
    h3 = o3[0:7, 0:7, 0:256]
    for t in range(1, 9):
        i, j = t // 3, t % 3
        h3 = h3 + o3[i:i + 7, j:j + 7, t * 256:(t + 1) * 256]
    h3 = jnp.maximum(h3 + b3_ref[...], 0.0).astype(bf16)

    t3 = h3.reshape(49, BB * C3_OC).T
    for c in range(C3_OC):
        out_ref[s * BB:(s + 1) * BB, c * 49:(c + 1) * 49] = \
            t3[c * BB:(c + 1) * BB, :]


def _head_kernel(f_ref, wh_ref, bh_ref, wo_ref, bo_ref, out_ref):
    f32 = jnp.float32
    hh = jnp.dot(f_ref[...], wh_ref[...], preferred_element_type=f32)
    hh = jnp.maximum(hh + bh_ref[...], 0.0).astype(jnp.bfloat16)
    out = jnp.dot(hh, wo_ref[...], preferred_element_type=f32) + bo_ref[...]
    lane = jax.lax.broadcasted_iota(jnp.int32, out.shape, 1)
    logits = jnp.where(lane < NA, out, -1e30)
    m = jnp.max(logits, axis=-1, keepdims=True)
    e = jnp.exp(logits - m)
    probs = e / jnp.sum(e, axis=-1, keepdims=True)
    out_ref[...] = jnp.where(lane == NA, out, probs)


def kernel(w1, b1, sel2, w2, b2, sel3, w3, b3, whead, bhead, wout, bout, x):
    del sel2, sel3
    f32, bf16 = jnp.float32, jnp.bfloat16
    B = x.shape[0]
    nblk = -(-B // (BB * SUB)) * SUB
    Bp = nblk * BB

    xb = x
    if Bp != B:
        xb = jnp.pad(xb, ((0, Bp - B), (0, 0), (0, 0), (0, 0)))

    xa = xb.reshape(nblk, BB, IN_C, HQ, 4, IN_W)
    xa = jnp.transpose(xa, (0, 3, 1, 2, 4, 5)).astype(bf16)
    xa = jnp.pad(xa, ((0, 0), (0, HQP - HQ), (0, 0), (0, 0), (0, 0),
                      (0, 4 * WQP - IN_W)))
    xph = xa.reshape(nblk, HQP, BB * IN_C * 4, 4 * WQP)

    w1c = w1.reshape(BB, IN_C, 2, 4, 2, 4, BB * C1_OC)
    w1c = w1c.transpose(5, 0, 1, 3, 2, 4, 6).reshape(4, BB * IN_C * 4,
                                                     4 * BB * C1_OC)

    tidx = jnp.array([[[[(2 * a + ry) * 4 + (2 * b + rx)
                         for b in range(2)] for a in range(2)]
                       for rx in range(2)] for ry in range(2)], jnp.int32)
    w2c = w2[tidx.reshape(-1)].reshape(2, 2, 2, 2, BB * C1_OC, BB * C2_OC)
    w2c = w2c.transpose(0, 1, 4, 2, 3, 5).reshape(4, BB * C1_OC,
                                                  4 * BB * C2_OC)

    w3c = w3.transpose(1, 0, 2).reshape(BB * C2_OC, 9 * BB * C3_OC)

    feat = pl.pallas_call(
        _conv_kernel,
        out_shape=jax.ShapeDtypeStruct((Bp, FEAT), bf16),
        grid=(nblk // SUB,),
        in_specs=[
            pl.BlockSpec((SUB, HQP, BB * IN_C * 4, 4 * WQP),
                         lambda i: (i, 0, 0, 0)),
            pl.BlockSpec(w1c.shape, lambda i: (0, 0, 0)),
            pl.BlockSpec(b1.shape, lambda i: (0, 0)),
            pl.BlockSpec(w2c.shape, lambda i: (0, 0, 0)),
            pl.BlockSpec(b2.shape, lambda i: (0, 0)),
            pl.BlockSpec(w3c.shape, lambda i: (0, 0)),
            pl.BlockSpec(b3.shape, lambda i: (0, 0)),
        ],
        out_specs=pl.BlockSpec((SUB * BB, FEAT), lambda i: (i, 0)),
        compiler_params=pltpu.CompilerParams(
            dimension_semantics=("parallel",),
            vmem_limit_bytes=64 * 1024 * 1024),
    )(xph, w1c, b1, w2c, b2, w3c, b3)

    HB = 128
    nhb = -(-Bp // HB)
    Bh = nhb * HB
    if Bh != Bp:
        feat = jnp.pad(feat, ((0, Bh - Bp), (0, 0)))
    y = pl.pallas_call(
        _head_kernel,
        out_shape=jax.ShapeDtypeStruct((Bh, HEAD_N), f32),
        grid=(nhb,),
        in_specs=[
            pl.BlockSpec((HB, FEAT), lambda i: (i, 0)),
            pl.BlockSpec(whead.shape, lambda i: (0, 0)),
            pl.BlockSpec(bhead.shape, lambda i: (0, 0)),
            pl.BlockSpec(wout.shape, lambda i: (0, 0)),
            pl.BlockSpec(bout.shape, lambda i: (0, 0)),
        ],
        out_specs=pl.BlockSpec((HB, HEAD_N), lambda i: (i, 0)),
        compiler_params=pltpu.CompilerParams(
            dimension_semantics=("parallel",),
            vmem_limit_bytes=64 * 1024 * 1024),
    )(feat, whead, bhead, wout, bout)

    probs = y[:B, :NA]
    value = y[:B, NA:NA + 1]
    return probs, value


# speedup vs baseline: 1.4476x; 1.1293x over previous
import jax
import jax.numpy as jnp
from jax.experimental import pallas as pl
from jax.experimental.pallas import tpu as pltpu

IN_C, IN_H, IN_W = 4, 84, 84
C1_OC, C2_OC, C3_OC = 16, 32, 32
HID = 256
HEAD_N = 128
NA = 6
BB = 8
FEAT = C3_OC * 49
HQ = IN_H // 4
HQP = 22
WQP = 24


SUB = 4


def _conv_kernel(x_ref, w1_ref, b1_ref, w2_ref, b2_ref, w3_ref, b3_ref,
                 out_ref):
    for s in range(SUB):
        _conv_block(x_ref[s], w1_ref, b1_ref, w2_ref, b2_ref, w3_ref, b3_ref,
                    out_ref, s)


def _conv_block(xs, w1_ref, b1_ref, w2_ref, b2_ref, w3_ref, b3_ref,
                out_ref, s):
    f32 = jnp.float32
    bf16 = jnp.bfloat16

    xb = xs.astype(bf16).reshape(BB * IN_C, HQ, 4, IN_W)
    xm = jnp.transpose(xb, (1, 0, 2, 3))
    xm = xm.reshape(HQ, BB * IN_C * 4, IN_W)
    xt = jnp.swapaxes(xm, 1, 2)
    xt = xt.reshape(HQ, HQ, 4, BB * IN_C * 4)
    o1 = None
    for wr in range(4):
        g = xt[:, :, wr, :].reshape(HQ * HQ, BB * IN_C * 4)
        d = jnp.dot(g, w1_ref[wr], preferred_element_type=f32)
        o1 = d if o1 is None else o1 + d
    o1 = o1.reshape(HQ, HQ, 4 * BB * C1_OC)
    h1 = (o1[0:20, 0:20, 0:128] + o1[0:20, 1:21, 128:256] +
          o1[1:21, 0:20, 256:384] + o1[1:21, 1:21, 384:512])
    h1 = jnp.maximum(h1 + b1_ref[...], 0.0).astype(bf16)

    h1p = h1.reshape(10, 2, 10, 2, BB * C1_OC)
    o2 = None
    for p, (ry, rx) in enumerate(((0, 0), (0, 1), (1, 0), (1, 1))):
        g = h1p[:, ry, :, rx, :].reshape(100, BB * C1_OC)
        d = jnp.dot(g, w2_ref[p], preferred_element_type=f32)
        o2 = d if o2 is None else o2 + d
    o2 = o2.reshape(10, 10, 4 * BB * C2_OC)
    h2 = (o2[0:9, 0:9, 0:256] + o2[0:9, 1:10, 256:512] +
          o2[1:10, 0:9, 512:768] + o2[1:10, 1:10, 768:1024])
    h2 = jnp.maximum(h2 + b2_ref[...], 0.0).astype(bf16)

    o3 = jnp.dot(h2.reshape(81, BB * C2_OC), w3_ref[...],
                 preferred_element_type=f32)
    o3 = o3.reshape(9, 9, 9 * BB * C3_OC)
    h3 = o3[0:7, 0:7, 0:256]
    for t in range(1, 9):
        i, j = t // 3, t % 3
        h3 = h3 + o3[i:i + 7, j:j + 7, t * 256:(t + 1) * 256]
    h3 = jnp.maximum(h3 + b3_ref[...], 0.0).astype(bf16)

    t3 = h3.reshape(49, BB * C3_OC).T
    for c in range(C3_OC):
        out_ref[s * BB:(s + 1) * BB, c * 49:(c + 1) * 49] = \
            t3[c * BB:(c + 1) * BB, :]


def _head_kernel(f_ref, wh_ref, bh_ref, wo_ref, bo_ref, out_ref):
    f32 = jnp.float32
    hh = jnp.dot(f_ref[...], wh_ref[...], preferred_element_type=f32)
    hh = jnp.maximum(hh + bh_ref[...], 0.0).astype(jnp.bfloat16)
    out = jnp.dot(hh, wo_ref[...], preferred_element_type=f32) + bo_ref[...]
    lane = jax.lax.broadcasted_iota(jnp.int32, out.shape, 1)
    logits = jnp.where(lane < NA, out, -1e30)
    m = jnp.max(logits, axis=-1, keepdims=True)
    e = jnp.exp(logits - m)
    probs = e / jnp.sum(e, axis=-1, keepdims=True)
    out_ref[...] = jnp.where(lane == NA, out, probs)


def kernel(w1, b1, sel2, w2, b2, sel3, w3, b3, whead, bhead, wout, bout, x):
    del sel2, sel3
    f32, bf16 = jnp.float32, jnp.bfloat16
    B = x.shape[0]
    nblk = -(-B // (BB * SUB)) * SUB
    Bp = nblk * BB

    xb = x
    if Bp != B:
        xb = jnp.pad(xb, ((0, Bp - B), (0, 0), (0, 0), (0, 0)))

    xph = xb.reshape(nblk, BB * IN_C, IN_H, IN_W)

    w1c = w1.reshape(BB, IN_C, 2, 4, 2, 4, BB * C1_OC)
    w1c = w1c.transpose(5, 0, 1, 3, 2, 4, 6).reshape(4, BB * IN_C * 4,
                                                     4 * BB * C1_OC)

    tidx = jnp.array([[[[(2 * a + ry) * 4 + (2 * b + rx)
                         for b in range(2)] for a in range(2)]
                       for rx in range(2)] for ry in range(2)], jnp.int32)
    w2c = w2[tidx.reshape(-1)].reshape(2, 2, 2, 2, BB * C1_OC, BB * C2_OC)
    w2c = w2c.transpose(0, 1, 4, 2, 3, 5).reshape(4, BB * C1_OC,
                                                  4 * BB * C2_OC)

    w3c = w3.transpose(1, 0, 2).reshape(BB * C2_OC, 9 * BB * C3_OC)

    feat = pl.pallas_call(
        _conv_kernel,
        out_shape=jax.ShapeDtypeStruct((Bp, FEAT), bf16),
        grid=(nblk // SUB,),
        in_specs=[
            pl.BlockSpec((SUB, BB * IN_C, IN_H, IN_W),
                         lambda i: (i, 0, 0, 0)),
            pl.BlockSpec(w1c.shape, lambda i: (0, 0, 0)),
            pl.BlockSpec(b1.shape, lambda i: (0, 0)),
            pl.BlockSpec(w2c.shape, lambda i: (0, 0, 0)),
            pl.BlockSpec(b2.shape, lambda i: (0, 0)),
            pl.BlockSpec(w3c.shape, lambda i: (0, 0)),
            pl.BlockSpec(b3.shape, lambda i: (0, 0)),
        ],
        out_specs=pl.BlockSpec((SUB * BB, FEAT), lambda i: (i, 0)),
        compiler_params=pltpu.CompilerParams(
            dimension_semantics=("parallel",),
            vmem_limit_bytes=64 * 1024 * 1024),
    )(xph, w1c, b1, w2c, b2, w3c, b3)

    HB = 128
    nhb = -(-Bp // HB)
    Bh = nhb * HB
    if Bh != Bp:
        feat = jnp.pad(feat, ((0, Bh - Bp), (0, 0)))
    y = pl.pallas_call(
        _head_kernel,
        out_shape=jax.ShapeDtypeStruct((Bh, HEAD_N), f32),
        grid=(nhb,),
        in_specs=[
            pl.BlockSpec((HB, FEAT), lambda i: (i, 0)),
            pl.BlockSpec(whead.shape, lambda i: (0, 0)),
            pl.BlockSpec(bhead.shape, lambda i: (0, 0)),
            pl.BlockSpec(wout.shape, lambda i: (0, 0)),
            pl.BlockSpec(bout.shape, lambda i: (0, 0)),
        ],
        out_specs=pl.BlockSpec((HB, HEAD_N), lambda i: (i, 0)),
        compiler_params=pltpu.CompilerParams(
            dimension_semantics=("parallel",),
            vmem_limit_bytes=64 * 1024 * 1024),
    )(feat, whead, bhead, wout, bout)

    probs = y[:B, :NA]
    value = y[:B, NA:NA + 1]
    return probs, value
